# baseline (speedup 1.0000x reference)
import functools

import jax
import jax.numpy as jnp
from jax import lax
from jax.experimental import pallas as pl
from jax.experimental.pallas import tpu as pltpu
from jax.experimental.pallas import tpu_sc as plsc

DIM = 64
BATCH = 16384
NC = 2
NS = 16
NW = NC * NS
CHUNK = BATCH // NW
S = 128
NSUB = CHUNK // S
NG = S // 16

def _sqrt16(v):
    x = jnp.maximum(v, jnp.float32(1e-30))
    i = plsc.bitcast(x, jnp.int32)
    y = plsc.bitcast(jnp.int32(0x5F3759DF) - (i >> 1), jnp.float32)
    for _ in range(3):
        y = y * (jnp.float32(1.5) - jnp.float32(0.5) * x * y * y)
    return x * y


def _make_kernel():
    mesh = plsc.VectorSubcoreMesh(core_axis_name="c", subcore_axis_name="s")
    fvmem = lambda shape: pltpu.VMEM(shape, jnp.float32)
    ivmem = lambda shape: pltpu.VMEM(shape, jnp.int32)

    scratch = (
        [ivmem((S,)) for _ in range(6)]
        + [fvmem((S, DIM)) for _ in range(8)]
        + [fvmem((S,))]
        + [pltpu.SemaphoreType.DMA, pltpu.SemaphoreType.DMA]
    )

    @functools.partial(
        pl.kernel,
        out_type=jax.ShapeDtypeStruct((BATCH,), jnp.float32),
        mesh=mesh,
        scratch_types=scratch,
        compiler_params=pltpu.CompilerParams(
            needs_layout_passes=False, use_tc_tiling_on_sc=False),
    )
    def trans_h(h_hbm, r_hbm, t_hbm, ent_hbm, rel_hbm, nrm_hbm, out_hbm,
                ih0, it0, ir0, ih1, it1, ir1,
                hb0, tb0, rb0, nb0, hb1, tb1, rb1, nb1,
                score_buf, sem0, sem1):
        wid = lax.axis_index("s") * NC + lax.axis_index("c")
        base = wid * CHUNK

        idx = ((ih0, it0, ir0), (ih1, it1, ir1))
        rows = ((hb0, tb0, rb0, nb0), (hb1, tb1, rb1, nb1))
        sems = (sem0, sem1)

        def fire(setid, sub):
            off = base + sub * S
            ih, it, ir = idx[setid]
            hb, tb, rb, nb = rows[setid]
            pltpu.sync_copy(h_hbm.at[pl.ds(off, S)], ih)
            pltpu.sync_copy(t_hbm.at[pl.ds(off, S)], it)
            pltpu.sync_copy(r_hbm.at[pl.ds(off, S)], ir)
            sem = sems[setid]
            return [
                pltpu.async_copy(ent_hbm.at[ih], hb, sem),
                pltpu.async_copy(ent_hbm.at[it], tb, sem),
                pltpu.async_copy(rel_hbm.at[ir], rb, sem),
                pltpu.async_copy(nrm_hbm.at[ir], nb, sem),
            ]

        def compute(setid, sub):
            off = base + sub * S
            hb, tb, rb, nb = rows[setid]

            def group(g, _):
                rix = g * 16 + lax.iota(jnp.int32, 16)
                zero = jnp.zeros((16,), jnp.float32)

                def dim_body(d, acc):
                    ww, wn, rn, nn = acc
                    col = jnp.full((16,), d, jnp.int32)
                    hv = plsc.load_gather(hb, [rix, col])
                    tv = plsc.load_gather(tb, [rix, col])
                    rv = plsc.load_gather(rb, [rix, col])
                    nv = plsc.load_gather(nb, [rix, col])
                    w = hv - tv + rv
                    return (ww + w * w, wn + w * nv, rn + rv * nv,
                            nn + nv * nv)

                ww, wn, rn, nn = lax.fori_loop(
                    0, DIM, dim_body, (zero, zero, zero, zero), unroll=8)
                c = (wn - rn) / jnp.maximum(nn, jnp.float32(1e-24))
                s2 = jnp.maximum(ww - 2.0 * c * wn + c * c * nn,
                                 jnp.float32(0.0))
                score_buf[pl.ds(g * 16, 16)] = _sqrt16(s2)
                return 0

            lax.fori_loop(0, NG, group, 0)
            pltpu.sync_copy(score_buf, out_hbm.at[pl.ds(off, S)])

        cps = fire(0, 0)
        for sub in range(NSUB):
            cur = sub & 1
            nxt_cps = fire(1 - cur, sub + 1) if sub + 1 < NSUB else None
            for c in cps:
                c.wait()
            compute(cur, sub)
            cps = nxt_cps

    return trans_h


_trans_h = _make_kernel()


@jax.jit
def kernel(h, r, t, entity_emb, relation_emb, relation_norm):
    return _trans_h(h.astype(jnp.int32), r.astype(jnp.int32),
                    t.astype(jnp.int32), entity_emb, relation_emb,
                    relation_norm)
